# baseline (speedup 1.0000x reference)
import functools

import jax
import jax.numpy as jnp
from jax import lax
from jax.experimental import pallas as pl
from jax.experimental.pallas import tpu as pltpu
from jax.experimental.pallas import tpu_sc as plsc

_NC = 2
_NS = 16
_NW = _NC * _NS
_C = 512


def _combined_body(tab_ref, out_ref):
    t = lax.broadcasted_iota(jnp.int32, (1, 168), 1)
    rows = lax.broadcasted_iota(jnp.int32, (32, 168), 0)
    oh = ((rows == t // 7) | (rows == (t // 24 + 24))).astype(jnp.float32)
    out_ref[...] = lax.dot_general(
        oh, tab_ref[...], (((0,), (0,)), ((), ())),
        preferred_element_type=jnp.float32,
        precision=lax.Precision.HIGHEST,
    )


def _make_sc_gather(n, d):
    b_per_w = n // _NW
    nchunks = b_per_w // _C
    mesh = plsc.VectorSubcoreMesh(core_axis_name="c", subcore_axis_name="s")

    @functools.partial(
        pl.kernel,
        mesh=mesh,
        out_type=jax.ShapeDtypeStruct((n, d), jnp.float32),
        scratch_types=[
            pltpu.VMEM((_C,), jnp.int32),
            pltpu.VMEM((_C, d), jnp.float32),
            pltpu.SemaphoreType.DMA,
        ],
    )
    def sc_gather(tab_hbm, idx_hbm, out_hbm, idx_v, rows_v, sem):
        wid = lax.axis_index("s") * _NC + lax.axis_index("c")
        w_base = wid * b_per_w

        def step(i, carry):
            base = pl.multiple_of(w_base + i * _C, _C)
            pltpu.sync_copy(idx_hbm.at[pl.ds(base, _C)], idx_v)
            pltpu.async_copy(tab_hbm.at[idx_v], rows_v, sem).wait()
            pltpu.sync_copy(rows_v, out_hbm.at[pl.ds(base, _C)])
            return carry

        lax.fori_loop(0, nchunks, step, 0)

    return sc_gather


_sc_gather_819200_128 = _make_sc_gather(819200, 128)


def kernel(time_index, hour_embed, weekday_embed):
    B, H = time_index.shape
    D = hour_embed.shape[1]
    N = B * H
    table32 = jnp.concatenate(
        [hour_embed, weekday_embed, jnp.zeros((1, D), jnp.float32)], axis=0)
    combined = pl.pallas_call(
        _combined_body,
        out_shape=jax.ShapeDtypeStruct((168, D), jnp.float32),
    )(table32)
    idx = time_index.reshape(N).astype(jnp.int32)
    out = _sc_gather_819200_128(combined, idx)
    return out.reshape(B, H, D)


# speedup vs baseline: 1.0123x; 1.0123x over previous
import functools

import jax
import jax.numpy as jnp
from jax import lax
from jax.experimental import pallas as pl
from jax.experimental.pallas import tpu as pltpu
from jax.experimental.pallas import tpu_sc as plsc

_NC = 2
_NS = 16
_NW = _NC * _NS
_C = 400


def _combined_body(tab_ref, out_ref):
    t = lax.broadcasted_iota(jnp.int32, (1, 168), 1)
    rows = lax.broadcasted_iota(jnp.int32, (32, 168), 0)
    oh = ((rows == t // 7) | (rows == (t // 24 + 24))).astype(jnp.float32)
    out_ref[...] = lax.dot_general(
        oh, tab_ref[...], (((0,), (0,)), ((), ())),
        preferred_element_type=jnp.float32,
        precision=lax.Precision.HIGHEST,
    )


def _make_sc_gather(n, d):
    b_per_w = n // _NW
    nchunks = b_per_w // _C
    assert b_per_w % _C == 0
    mesh = plsc.VectorSubcoreMesh(core_axis_name="c", subcore_axis_name="s")

    @functools.partial(
        pl.kernel,
        mesh=mesh,
        out_type=jax.ShapeDtypeStruct((n, d), jnp.float32),
        scratch_types=[
            pltpu.VMEM((b_per_w,), jnp.int32),
            pltpu.VMEM((2, _C, d), jnp.float32),
            pltpu.SemaphoreType.DMA,
            pltpu.SemaphoreType.DMA,
            pltpu.SemaphoreType.DMA,
            pltpu.SemaphoreType.DMA,
        ],
    )
    def sc_gather(tab_hbm, idx_hbm, out_hbm, idx_v, rows_v, g0, g1, w0, w1):
        wid = lax.axis_index("s") * _NC + lax.axis_index("c")
        w_base = wid * b_per_w
        pltpu.sync_copy(idx_hbm.at[pl.ds(w_base, b_per_w)], idx_v)
        gsem = (g0, g1)
        wsem = (w0, w1)

        def start_gather(i):
            b = i % 2
            return pltpu.async_copy(
                tab_hbm.at[idx_v.at[pl.ds(i * _C, _C)]], rows_v.at[b], gsem[b])

        gcp = [start_gather(0), None]
        wcp = [None, None]
        for i in range(nchunks):
            b = i % 2
            gcp[b].wait()
            wcp[b] = pltpu.async_copy(
                rows_v.at[b], out_hbm.at[pl.ds(w_base + i * _C, _C)], wsem[b])
            if i + 1 < nchunks:
                if i >= 1:
                    wcp[1 - b].wait()
                gcp[1 - b] = start_gather(i + 1)
        wcp[0].wait()
        wcp[1].wait()

    return sc_gather


_sc_gather_819200_128 = _make_sc_gather(819200, 128)


def kernel(time_index, hour_embed, weekday_embed):
    B, H = time_index.shape
    D = hour_embed.shape[1]
    N = B * H
    table32 = jnp.concatenate(
        [hour_embed, weekday_embed, jnp.zeros((1, D), jnp.float32)], axis=0)
    combined = pl.pallas_call(
        _combined_body,
        out_shape=jax.ShapeDtypeStruct((168, D), jnp.float32),
    )(table32)
    idx = time_index.reshape(N).astype(jnp.int32)
    out = _sc_gather_819200_128(combined, idx)
    return out.reshape(B, H, D)


# speedup vs baseline: 4.6667x; 4.6100x over previous
import functools

import jax
import jax.numpy as jnp
from jax import lax
from jax.experimental import pallas as pl
from jax.experimental.pallas import tpu as pltpu
from jax.experimental.pallas import tpu_sc as plsc

_NC = 2
_NS = 16
_NW = _NC * _NS
_C = 400


def _combined_body(tab_ref, out_ref):
    t = lax.broadcasted_iota(jnp.int32, (1, 168), 1)
    rows = lax.broadcasted_iota(jnp.int32, (32, 168), 0)
    oh = ((rows == t // 7) | (rows == (t // 24 + 24))).astype(jnp.float32)
    out_ref[...] = lax.dot_general(
        oh, tab_ref[...], (((0,), (0,)), ((), ())),
        preferred_element_type=jnp.float32,
        precision=lax.Precision.HIGHEST,
    )


def _make_sc_gather(n, d):
    b_per_w = n // _NW
    nchunks = b_per_w // _C
    assert b_per_w % _C == 0
    mesh = plsc.VectorSubcoreMesh(core_axis_name="c", subcore_axis_name="s")

    @functools.partial(
        pl.kernel,
        mesh=mesh,
        out_type=jax.ShapeDtypeStruct((n, d), jnp.float32),
        scratch_types=[
            pltpu.VMEM((b_per_w,), jnp.int32),
            pltpu.VMEM((2, _C, d), jnp.float32),
            pltpu.VMEM_SHARED((168, d), jnp.float32),
            pltpu.SemaphoreType.DMA,
            pltpu.SemaphoreType.DMA,
            pltpu.SemaphoreType.DMA,
            pltpu.SemaphoreType.DMA,
        ],
    )
    def sc_gather(tab_hbm, idx_hbm, out_hbm, idx_v, rows_v, tab_sh,
                  g0, g1, w0, w1):
        sid = lax.axis_index("s")
        wid = sid * _NC + lax.axis_index("c")
        w_base = wid * b_per_w
        @pl.when(sid == 0)
        def _():
            pltpu.sync_copy(tab_hbm, tab_sh)
        pltpu.sync_copy(idx_hbm.at[pl.ds(w_base, b_per_w)], idx_v)
        plsc.subcore_barrier()
        gsem = (g0, g1)
        wsem = (w0, w1)

        def start_gather(i):
            b = i % 2
            return pltpu.async_copy(
                tab_sh.at[idx_v.at[pl.ds(i * _C, _C)]], rows_v.at[b], gsem[b])

        gcp = [start_gather(0), None]
        wcp = [None, None]
        for i in range(nchunks):
            b = i % 2
            gcp[b].wait()
            wcp[b] = pltpu.async_copy(
                rows_v.at[b], out_hbm.at[pl.ds(w_base + i * _C, _C)], wsem[b])
            if i + 1 < nchunks:
                if i >= 1:
                    wcp[1 - b].wait()
                gcp[1 - b] = start_gather(i + 1)
        wcp[0].wait()
        wcp[1].wait()

    return sc_gather


_sc_gather_819200_128 = _make_sc_gather(819200, 128)


def kernel(time_index, hour_embed, weekday_embed):
    B, H = time_index.shape
    D = hour_embed.shape[1]
    N = B * H
    table32 = jnp.concatenate(
        [hour_embed, weekday_embed, jnp.zeros((1, D), jnp.float32)], axis=0)
    combined = pl.pallas_call(
        _combined_body,
        out_shape=jax.ShapeDtypeStruct((168, D), jnp.float32),
    )(table32)
    idx = time_index.reshape(N).astype(jnp.int32)
    out = _sc_gather_819200_128(combined, idx)
    return out.reshape(B, H, D)
